# baseline (speedup 1.0000x reference)
import jax
import jax.numpy as jnp
from jax.experimental import pallas as pl
from jax.experimental.pallas import tpu as pltpu

HIDDEN = 256
NVOC = 5
NAUX = 60
ROWS_PER_BLOCK = 2048


def _body(ids_ref, aux_ref, out_ref):
    r = out_ref.shape[0]
    ids = ids_ref[...]
    aux = aux_ref[...]
    col = jax.lax.broadcasted_iota(jnp.int32, (r, HIDDEN), 1)
    oh = jnp.where((col == ids) & (col < NVOC), 1.0, 0.0).astype(jnp.float32)
    z_left = jnp.zeros((r, NVOC), jnp.float32)
    z_right = jnp.zeros((r, HIDDEN - NVOC - NAUX), jnp.float32)
    shifted = jnp.concatenate([z_left, aux, z_right], axis=-1)
    out_ref[...] = oh + shifted


def kernel(input_ids, aux_features):
    b, s = input_ids.shape
    n = b * s
    ids2d = input_ids.reshape(n, 1).astype(jnp.int32)
    aux2d = aux_features.reshape(n, NAUX)
    grid = (n // ROWS_PER_BLOCK,)
    out = pl.pallas_call(
        _body,
        grid=grid,
        in_specs=[
            pl.BlockSpec((ROWS_PER_BLOCK, 1), lambda i: (i, 0)),
            pl.BlockSpec((ROWS_PER_BLOCK, NAUX), lambda i: (i, 0)),
        ],
        out_specs=pl.BlockSpec((ROWS_PER_BLOCK, HIDDEN), lambda i: (i, 0)),
        out_shape=jax.ShapeDtypeStruct((n, HIDDEN), jnp.float32),
    )(ids2d, aux2d)
    return out.reshape(b, s, HIDDEN)


# speedup vs baseline: 1.7568x; 1.7568x over previous
import jax
import jax.numpy as jnp
from jax.experimental import pallas as pl
from jax.experimental.pallas import tpu as pltpu

HIDDEN = 256
NVOC = 5
NAUX = 60
BATCH_BLOCK = 8


def _body(ids_ref, aux_ref, out_ref):
    g, s = ids_ref.shape
    ids = ids_ref[...][:, :, None]
    aux = aux_ref[...]
    col = jax.lax.broadcasted_iota(jnp.int32, (g, s, HIDDEN), 2)
    oh = jnp.where((col == ids) & (col < NVOC), 1.0, 0.0).astype(jnp.float32)
    z_left = jnp.zeros((g, s, NVOC), jnp.float32)
    z_right = jnp.zeros((g, s, HIDDEN - NVOC - NAUX), jnp.float32)
    shifted = jnp.concatenate([z_left, aux, z_right], axis=-1)
    out_ref[...] = oh + shifted


def kernel(input_ids, aux_features):
    b, s = input_ids.shape
    g = BATCH_BLOCK
    return pl.pallas_call(
        _body,
        grid=(b // g,),
        in_specs=[
            pl.BlockSpec((g, s), lambda i: (i, 0)),
            pl.BlockSpec((g, s, NAUX), lambda i: (i, 0, 0)),
        ],
        out_specs=pl.BlockSpec((g, s, HIDDEN), lambda i: (i, 0, 0)),
        out_shape=jax.ShapeDtypeStruct((b, s, HIDDEN), jnp.float32),
    )(input_ids, aux_features)


# speedup vs baseline: 1.7894x; 1.0185x over previous
import jax
import jax.numpy as jnp
from jax.experimental import pallas as pl
from jax.experimental.pallas import tpu as pltpu

HIDDEN = 256
NVOC = 5
NAUX = 60
BATCH_BLOCK = 16


def _body(ids_ref, aux_ref, out_ref):
    g, s = ids_ref.shape
    ids = ids_ref[...][:, :, None]
    aux = aux_ref[...]
    col = jax.lax.broadcasted_iota(jnp.int32, (g, s, HIDDEN), 2)
    oh = jnp.where((col == ids) & (col < NVOC), 1.0, 0.0).astype(jnp.float32)
    z_left = jnp.zeros((g, s, NVOC), jnp.float32)
    z_right = jnp.zeros((g, s, HIDDEN - NVOC - NAUX), jnp.float32)
    shifted = jnp.concatenate([z_left, aux, z_right], axis=-1)
    out_ref[...] = oh + shifted


def kernel(input_ids, aux_features):
    b, s = input_ids.shape
    g = BATCH_BLOCK
    return pl.pallas_call(
        _body,
        grid=(b // g,),
        in_specs=[
            pl.BlockSpec((g, s), lambda i: (i, 0)),
            pl.BlockSpec((g, s, NAUX), lambda i: (i, 0, 0)),
        ],
        out_specs=pl.BlockSpec((g, s, HIDDEN), lambda i: (i, 0, 0)),
        out_shape=jax.ShapeDtypeStruct((b, s, HIDDEN), jnp.float32),
    )(input_ids, aux_features)
